# Initial kernel scaffold; baseline (speedup 1.0000x reference)
import jax
import jax.numpy as jnp
from jax.experimental import pallas as pl


def kernel(W):
    raise NotImplementedError("write your pallas kernel here")


# speedup vs baseline: 3.6702x; 3.6702x over previous
import functools

import jax
import jax.numpy as jnp
from jax.experimental import pallas as pl

TOP_K = 32


def _block_kernel(w_rows_ref, w_ref, out_ref, *, k):
    w_rows = w_rows_ref[...]
    w = w_ref[...]
    sim = jax.lax.dot_general(
        w_rows, w,
        dimension_numbers=(((1,), (1,)), ((), ())),
        preferred_element_type=jnp.float32,
    )

    n = sim.shape[1]
    iota = jax.lax.broadcasted_iota(jnp.int32, sim.shape, 1)
    big = jnp.int32(n + 1)

    cur = sim
    keep = jnp.zeros(sim.shape, dtype=jnp.bool_)
    acc = jnp.zeros((sim.shape[0], 1), dtype=jnp.float32)
    for _ in range(k):
        m = jnp.max(cur, axis=-1, keepdims=True)
        idx = jnp.min(jnp.where(cur == m, iota, big), axis=-1, keepdims=True)
        sel = iota == idx
        keep = jnp.logical_or(keep, sel)
        cur = jnp.where(sel, -jnp.inf, cur)
        acc = acc + m * m

    rnorm = 1.0 / jnp.maximum(jnp.sqrt(acc), 1e-12)
    out_ref[...] = jnp.where(keep, sim * rnorm, 0.0)


def kernel(W):
    n, d = W.shape
    r = 200 if n % 200 == 0 else n
    grid = n // r
    return pl.pallas_call(
        functools.partial(_block_kernel, k=TOP_K),
        grid=(grid,),
        in_specs=[
            pl.BlockSpec((r, d), lambda i: (i, 0)),
            pl.BlockSpec((n, d), lambda i: (0, 0)),
        ],
        out_specs=pl.BlockSpec((r, n), lambda i: (i, 0)),
        out_shape=jax.ShapeDtypeStruct((n, n), jnp.float32),
    )(W, W)


# speedup vs baseline: 5.3258x; 1.4511x over previous
import functools

import jax
import jax.numpy as jnp
from jax.experimental import pallas as pl
from jax.experimental.pallas import tpu as pltpu

TOP_K = 32


def _block_kernel(w_rows_ref, w_ref, out_ref, *, k):
    w_rows = w_rows_ref[...]
    w = w_ref[...]
    sim = jax.lax.dot_general(
        w_rows, w,
        dimension_numbers=(((1,), (1,)), ((), ())),
        preferred_element_type=jnp.float32,
    )

    n = sim.shape[1]
    iota = jax.lax.broadcasted_iota(jnp.int32, sim.shape, 1)
    big = jnp.int32(n + 1)

    cur = sim
    for _ in range(k):
        m = jnp.max(cur, axis=-1, keepdims=True)
        idx = jnp.min(jnp.where(cur == m, iota, big), axis=-1, keepdims=True)
        cur = jnp.where(iota == idx, -jnp.inf, cur)

    keep = cur == -jnp.inf
    vals = jnp.where(keep, sim, 0.0)
    acc = jnp.sum(vals * vals, axis=-1, keepdims=True)
    rnorm = 1.0 / jnp.maximum(jnp.sqrt(acc), 1e-12)
    out_ref[...] = vals * rnorm


def kernel(W):
    n, d = W.shape
    r = 200 if n % 200 == 0 else n
    grid = n // r
    return pl.pallas_call(
        functools.partial(_block_kernel, k=TOP_K),
        grid=(grid,),
        in_specs=[
            pl.BlockSpec((r, d), lambda i: (i, 0)),
            pl.BlockSpec((n, d), lambda i: (0, 0)),
        ],
        out_specs=pl.BlockSpec((r, n), lambda i: (i, 0)),
        out_shape=jax.ShapeDtypeStruct((n, n), jnp.float32),
        compiler_params=pltpu.CompilerParams(
            dimension_semantics=("parallel",),
        ),
    )(W, W)


# speedup vs baseline: 12.6151x; 2.3687x over previous
import functools

import jax
import jax.numpy as jnp
from jax.experimental import pallas as pl
from jax.experimental.pallas import tpu as pltpu

TOP_K = 32
_MSB_INT = -2147483648


def _block_kernel(w_rows_ref, w_ref, out_ref, *, k):
    w_rows = w_rows_ref[...]
    w = w_ref[...]
    sim = jax.lax.dot_general(
        w_rows, w,
        dimension_numbers=(((1,), (1,)), ((), ())),
        preferred_element_type=jnp.float32,
    )

    n = sim.shape[1]
    kk = jnp.int32(k)
    _MSB = jnp.int32(_MSB_INT)

    b = jax.lax.bitcast_convert_type(sim, jnp.int32)
    s = jnp.where(b < 0, _MSB - b, b)

    p = jnp.zeros((sim.shape[0], 1), dtype=jnp.int32)
    for bit in range(31, -1, -1):
        cand = p | (jnp.int32(1) << jnp.int32(bit))
        thr = cand ^ _MSB
        cnt = jnp.sum(jnp.where(s >= thr, jnp.int32(1), jnp.int32(0)),
                      axis=-1, keepdims=True)
        p = jnp.where(cnt >= kk, cand, p)
    tstar = p ^ _MSB

    gt = s > tstar
    eq = s == tstar
    n_gt = jnp.sum(jnp.where(gt, jnp.int32(1), jnp.int32(0)),
                   axis=-1, keepdims=True)
    n_eq = jnp.sum(jnp.where(eq, jnp.int32(1), jnp.int32(0)),
                   axis=-1, keepdims=True)
    extra = kk - n_gt

    iota = jax.lax.broadcasted_iota(jnp.int32, sim.shape, 1)
    last = jnp.int32(n - 1)
    lo0 = jnp.where(n_eq == extra, last, jnp.int32(0))
    hi0 = jnp.broadcast_to(last, lo0.shape)

    def cond(carry):
        lo, hi = carry
        return jnp.any(lo < hi)

    def body(carry):
        lo, hi = carry
        mid = lo + (hi - lo) // 2
        c = jnp.sum(jnp.where(eq & (iota <= mid), jnp.int32(1),
                              jnp.int32(0)), axis=-1, keepdims=True)
        take = c >= extra
        return jnp.where(take, lo, mid + 1), jnp.where(take, mid, hi)

    _, jidx = jax.lax.while_loop(cond, body, (lo0, hi0))

    keep = gt | (eq & (iota <= jidx))
    vals = jnp.where(keep, sim, 0.0)
    acc = jnp.sum(vals * vals, axis=-1, keepdims=True)
    rnorm = 1.0 / jnp.maximum(jnp.sqrt(acc), 1e-12)
    out_ref[...] = vals * rnorm


def kernel(W):
    n, d = W.shape
    r = 200 if n % 200 == 0 else n
    grid = n // r
    return pl.pallas_call(
        functools.partial(_block_kernel, k=TOP_K),
        grid=(grid,),
        in_specs=[
            pl.BlockSpec((r, d), lambda i: (i, 0)),
            pl.BlockSpec((n, d), lambda i: (0, 0)),
        ],
        out_specs=pl.BlockSpec((r, n), lambda i: (i, 0)),
        out_shape=jax.ShapeDtypeStruct((n, n), jnp.float32),
        compiler_params=pltpu.CompilerParams(
            dimension_semantics=("parallel",),
        ),
    )(W, W)
